# Initial kernel scaffold; baseline (speedup 1.0000x reference)
import jax
import jax.numpy as jnp
from jax.experimental import pallas as pl


def kernel(query, thresholds, keys_0, keys_1, keys_2, keys_3, values_0, values_1, values_2, values_3):
    raise NotImplementedError("write your pallas kernel here")


# speedup vs baseline: 1.1518x; 1.1518x over previous
import jax
import jax.numpy as jnp
from jax.experimental import pallas as pl
from jax.experimental.pallas import tpu as pltpu

_B, _S, _D, _M, _P = 4096, 4, 512, 5000, 336
_TEMP = 0.07
_GATE_SHARP = 10.0
_BB = 256


def _knorm_body(k_ref, o_ref):
    k = k_ref[0]
    o_ref[0, 0] = 1.0 / (jnp.sqrt(jnp.sum(k * k, axis=1)) + 1e-8)


def _main_body(thr_ref, q_ref, k_ref, v_ref, kinv_ref, o_ref):
    s = pl.program_id(0)
    q = q_ref[0]
    qinv = 1.0 / (jnp.sqrt(jnp.sum(q * q, axis=1, keepdims=True)) + 1e-8)
    raw = jax.lax.dot_general(
        q, k_ref[0], (((1,), (1,)), ((), ())),
        preferred_element_type=jnp.float32)
    sim = raw * qinv * kinv_ref[0]
    m = jnp.max(sim, axis=1, keepdims=True)
    e = jnp.exp((sim - m) * (1.0 / _TEMP))
    gate = jax.nn.sigmoid((sim - thr_ref[s]) * _GATE_SHARP)
    eg = e * gate
    z = jnp.sum(e, axis=1, keepdims=True)
    zg = jnp.sum(eg, axis=1, keepdims=True)
    num = jax.lax.dot_general(
        eg, v_ref[0], (((1,), (0,)), ((), ())),
        preferred_element_type=jnp.float32)
    o_ref[0] = num / (zg + 1e-8 * z)


@jax.jit
def kernel(query, thresholds, keys_0, keys_1, keys_2, keys_3,
           values_0, values_1, values_2, values_3):
    K = jnp.stack([keys_0, keys_1, keys_2, keys_3])
    V = jnp.stack([values_0, values_1, values_2, values_3])
    qT = jnp.transpose(query, (1, 0, 2))

    kinv = pl.pallas_call(
        _knorm_body,
        grid=(_S,),
        in_specs=[pl.BlockSpec((1, _M, _D), lambda s: (s, 0, 0))],
        out_specs=pl.BlockSpec((1, 1, _M), lambda s: (s, 0, 0)),
        out_shape=jax.ShapeDtypeStruct((_S, 1, _M), jnp.float32),
    )(K)

    out = pl.pallas_call(
        _main_body,
        grid=(_S, _B // _BB),
        in_specs=[
            pl.BlockSpec(memory_space=pltpu.SMEM),
            pl.BlockSpec((1, _BB, _D), lambda s, i: (s, i, 0)),
            pl.BlockSpec((1, _M, _D), lambda s, i: (s, 0, 0)),
            pl.BlockSpec((1, _M, _P), lambda s, i: (s, 0, 0)),
            pl.BlockSpec((1, 1, _M), lambda s, i: (s, 0, 0)),
        ],
        out_specs=pl.BlockSpec((1, _BB, _P), lambda s, i: (s, i, 0)),
        out_shape=jax.ShapeDtypeStruct((_S, _B, _P), jnp.float32),
    )(thresholds, qT, K, V, kinv)

    return jnp.transpose(out, (1, 0, 2))


# speedup vs baseline: 1.6661x; 1.4465x over previous
import jax
import jax.numpy as jnp
from jax.experimental import pallas as pl
from jax.experimental.pallas import tpu as pltpu

_B, _S, _D, _M, _P = 4096, 4, 512, 5000, 336
_TEMP = 0.07
_GATE_SHARP = 10.0
_BB = 256


def _knorm_body(k_ref, o_ref):
    k = k_ref[0]
    o_ref[0, 0] = 1.0 / (jnp.sqrt(jnp.sum(k * k, axis=1)) + 1e-8)


def _main_body(thr_ref, q_ref, k_ref, v_ref, kinv_ref, o_ref):
    s = pl.program_id(0)
    q = q_ref[0]
    qs = (1.0 / _TEMP) / (jnp.sqrt(jnp.sum(q * q, axis=1, keepdims=True)) + 1e-8)
    raw = jax.lax.dot_general(
        q, k_ref[0], (((1,), (1,)), ((), ())),
        preferred_element_type=jnp.float32)
    x = raw * qs * kinv_ref[0]
    e = jnp.exp(x)
    gate = jax.nn.sigmoid(x * (_GATE_SHARP * _TEMP) - thr_ref[s] * _GATE_SHARP)
    eg = e * gate
    z = jnp.sum(e, axis=1, keepdims=True)
    numa = jax.lax.dot_general(
        eg.astype(jnp.bfloat16), v_ref[0], (((1,), (0,)), ((), ())),
        preferred_element_type=jnp.float32)
    zg = numa[:, _P:]
    o_ref[0] = numa[:, :_P] / (zg + 1e-8 * z)


@jax.jit
def kernel(query, thresholds, keys_0, keys_1, keys_2, keys_3,
           values_0, values_1, values_2, values_3):
    K = jnp.stack([keys_0, keys_1, keys_2, keys_3])
    V = jnp.stack([values_0, values_1, values_2, values_3])
    Va = jnp.concatenate(
        [V, jnp.ones((_S, _M, 1), V.dtype)], axis=2).astype(jnp.bfloat16)
    qT = jnp.transpose(query, (1, 0, 2))

    kinv = pl.pallas_call(
        _knorm_body,
        grid=(_S,),
        in_specs=[pl.BlockSpec((1, _M, _D), lambda s: (s, 0, 0))],
        out_specs=pl.BlockSpec((1, 1, _M), lambda s: (s, 0, 0)),
        out_shape=jax.ShapeDtypeStruct((_S, 1, _M), jnp.float32),
    )(K)

    out = pl.pallas_call(
        _main_body,
        grid=(_S, _B // _BB),
        in_specs=[
            pl.BlockSpec(memory_space=pltpu.SMEM),
            pl.BlockSpec((1, _BB, _D), lambda s, i: (s, i, 0)),
            pl.BlockSpec((1, _M, _D), lambda s, i: (s, 0, 0)),
            pl.BlockSpec((1, _M, _P + 1), lambda s, i: (s, 0, 0)),
            pl.BlockSpec((1, 1, _M), lambda s, i: (s, 0, 0)),
        ],
        out_specs=pl.BlockSpec((1, _BB, _P), lambda s, i: (s, i, 0)),
        out_shape=jax.ShapeDtypeStruct((_S, _B, _P), jnp.float32),
    )(thresholds, qT, K, Va, kinv)

    return jnp.transpose(out, (1, 0, 2))


# speedup vs baseline: 1.6987x; 1.0196x over previous
import functools

import jax
import jax.numpy as jnp
from jax.experimental import pallas as pl
from jax.experimental.pallas import tpu as pltpu

_B, _S, _D, _M, _P = 4096, 4, 512, 5000, 336
_TEMP = 0.07
_GATE_SHARP = 10.0
_BB = 256


def _make_body(s, has_prev):
    def body(*refs):
        thr_ref, q_ref, k_ref, v_ref, o_ref = refs[:5]
        kinv_scr, vb_scr = refs[-2:]

        i = pl.program_id(0)

        @pl.when(i == 0)
        def _init():
            k = k_ref[...]
            s2 = jax.lax.dot_general(
                jnp.ones((8, _D), jnp.float32), k * k,
                (((1,), (1,)), ((), ())),
                preferred_element_type=jnp.float32)
            kinv_scr[...] = 1.0 / (jnp.sqrt(s2[:1]) + 1e-8)
            vb_scr[:, :_P] = v_ref[...].astype(jnp.bfloat16)
            vb_scr[:, _P:] = jnp.ones((_M, 1), jnp.bfloat16)

        q = q_ref[...]
        qs = (1.0 / _TEMP) / (
            jnp.sqrt(jnp.sum(q * q, axis=1, keepdims=True)) + 1e-8)
        raw = jax.lax.dot_general(
            q, k_ref[...], (((1,), (1,)), ((), ())),
            preferred_element_type=jnp.float32)
        x = raw * qs * kinv_scr[...]
        e = jnp.exp(x)
        gate = jax.nn.sigmoid(
            x * (_GATE_SHARP * _TEMP) - thr_ref[s] * _GATE_SHARP)
        eg = e * gate
        z = jnp.sum(e, axis=1, keepdims=True)
        numa = jax.lax.dot_general(
            eg.astype(jnp.bfloat16), vb_scr[...], (((1,), (0,)), ((), ())),
            preferred_element_type=jnp.float32)
        o_ref[...] = numa[:, :_P] / (numa[:, _P:] + 1e-8 * z)

    return body


@jax.jit
def kernel(query, thresholds, keys_0, keys_1, keys_2, keys_3,
           values_0, values_1, values_2, values_3):
    keys = [keys_0, keys_1, keys_2, keys_3]
    values = [values_0, values_1, values_2, values_3]
    q2 = query.reshape(_B, _S * _D)

    outs = []
    for s in range(_S):
        in_specs = [
            pl.BlockSpec(memory_space=pltpu.SMEM),
            pl.BlockSpec((_BB, _D), functools.partial(
                lambda i, s=s: (i, s))),
            pl.BlockSpec((_M, _D), lambda i: (0, 0)),
            pl.BlockSpec((_M, _P), lambda i: (0, 0)),
        ]
        outs.append(pl.pallas_call(
            _make_body(s, False),
            grid=(_B // _BB,),
            in_specs=in_specs,
            out_specs=pl.BlockSpec((_BB, _P), lambda i: (i, 0)),
            out_shape=jax.ShapeDtypeStruct((_B, _P), jnp.float32),
            scratch_shapes=[
                pltpu.VMEM((1, _M), jnp.float32),
                pltpu.VMEM((_M, _P + 1), jnp.bfloat16),
            ],
        )(thresholds, q2, keys[s], values[s]))

    return jnp.stack(outs, axis=1)


# speedup vs baseline: 1.9647x; 1.1566x over previous
import jax
import jax.numpy as jnp
from jax.experimental import pallas as pl
from jax.experimental.pallas import tpu as pltpu

_B, _S, _D, _M, _P = 4096, 4, 512, 5000, 336
_TEMP = 0.07
_GATE_SHARP = 10.0
_BB = 256
_SC = 2


def _make_body(scales):
    def body(thr_ref, q_ref, ka_ref, kb_ref, vta_ref, vtb_ref, o_ref,
             kinva, kinvb, vbta, vbtb):
        i = pl.program_id(0)
        k_refs = [ka_ref, kb_ref]
        vt_refs = [vta_ref, vtb_ref]
        kinvs = [kinva, kinvb]
        vbts = [vbta, vbtb]

        @pl.when(i == 0)
        def _init():
            for j in range(_SC):
                k = k_refs[j][...]
                s2 = jax.lax.dot_general(
                    jnp.ones((8, _D), jnp.float32), k * k,
                    (((1,), (1,)), ((), ())),
                    preferred_element_type=jnp.float32)
                kinvs[j][...] = 1.0 / (jnp.sqrt(s2[:1]) + 1e-8)
                vbts[j][:_P, :] = vt_refs[j][...].astype(jnp.bfloat16)
                vbts[j][_P:, :] = jnp.ones((1, _M), jnp.bfloat16)

        pieces = []
        for j, s in enumerate(scales):
            q = q_ref[:, s, :]
            qs = (1.0 / _TEMP) / (
                jnp.sqrt(jnp.sum(q * q, axis=1, keepdims=True)) + 1e-8)
            raw = jax.lax.dot_general(
                q, k_refs[j][...], (((1,), (1,)), ((), ())),
                preferred_element_type=jnp.float32)
            x = raw * qs * kinvs[j][...]
            e = jnp.exp(x)
            gate = jax.nn.sigmoid(
                x * (_GATE_SHARP * _TEMP) - thr_ref[s] * _GATE_SHARP)
            eg = e * gate
            z = jnp.sum(e, axis=1, keepdims=True)
            numa = jax.lax.dot_general(
                eg.astype(jnp.bfloat16), vbts[j][...],
                (((1,), (1,)), ((), ())),
                preferred_element_type=jnp.float32)
            pieces.append(numa[:, :_P] / (numa[:, _P:] + 1e-8 * z))
        o_ref[...] = jnp.concatenate(pieces, axis=1)

    return body


@jax.jit
def kernel(query, thresholds, keys_0, keys_1, keys_2, keys_3,
           values_0, values_1, values_2, values_3):
    keys = [keys_0, keys_1, keys_2, keys_3]
    values = [values_0, values_1, values_2, values_3]
    const = lambda i: (0, 0)

    halves = []
    for scales in ((0, 1), (2, 3)):
        out = pl.pallas_call(
            _make_body(scales),
            grid=(_B // _BB,),
            in_specs=[
                pl.BlockSpec(memory_space=pltpu.SMEM),
                pl.BlockSpec((_BB, _S, _D), lambda i: (i, 0, 0)),
                pl.BlockSpec((_M, _D), const),
                pl.BlockSpec((_M, _D), const),
                pl.BlockSpec((_P, _M), const),
                pl.BlockSpec((_P, _M), const),
            ],
            out_specs=pl.BlockSpec((_BB, _SC * _P), lambda i: (i, 0)),
            out_shape=jax.ShapeDtypeStruct((_B, _SC * _P), jnp.float32),
            scratch_shapes=(
                [pltpu.VMEM((1, _M), jnp.float32) for _ in range(_SC)]
                + [pltpu.VMEM((_P + 1, _M), jnp.bfloat16) for _ in range(_SC)]
            ),
        )(thresholds, query, keys[scales[0]], keys[scales[1]],
          values[scales[0]].T, values[scales[1]].T)
        halves.append(out.reshape(_B, _SC, _P))

    return jnp.concatenate(halves, axis=1)
